# baseline (speedup 1.0000x reference)
import functools

import jax
import jax.numpy as jnp
import numpy as np
from jax import lax
from jax.experimental import pallas as pl
from jax.experimental.pallas import tpu as pltpu
from jax.experimental.pallas import tpu_sc as plsc

N = 10000
E = 320000
F_IN = 128
H1 = 8
C1 = 64
D1 = H1 * C1
C2 = 64
G = 64
NCLS = 2

NSC = 2
NSUB = 16

ROW_BLK = 2000

NP = 10240
ROW_BLK1 = 2048
W1 = D1 + 16
NPASS1 = 4
QN = NP // (NSC * NPASS1)
EPT1 = E // NSUB
SCAN_CH = 4000
NSCAN = EPT1 // SCAN_CH
KB1 = 32
CAP1 = SCAN_CH + 16
WS = D1
WD = D1 + 32

_PI = (np.arange(0, D1, 32)[:, None]
       + np.concatenate([np.arange(0, 32, 2), np.arange(1, 32, 2)])[None, :]
       ).reshape(-1)

W2 = 80
EPT2 = E // (NSC * NSUB)
KB2 = 80
NCH2 = EPT2 // KB2

_SC_MESH = plsc.VectorSubcoreMesh(
    core_axis_name="c", subcore_axis_name="s", num_cores=NSC, num_subcores=NSUB)


def _mm2_body(a_ref, w1_ref, w2_ref, o1_ref, o2_ref):
    a = a_ref[...]
    o1_ref[...] = jnp.dot(a, w1_ref[...], preferred_element_type=jnp.float32)
    o2_ref[...] = jnp.dot(a, w2_ref[...], preferred_element_type=jnp.float32)


def _mm2(a, w1, w2):
    n, k = a.shape
    m = w1.shape[1]
    return pl.pallas_call(
        _mm2_body,
        grid=(n // ROW_BLK,),
        in_specs=[
            pl.BlockSpec((ROW_BLK, k), lambda i: (i, 0)),
            pl.BlockSpec((k, m), lambda i: (0, 0)),
            pl.BlockSpec((k, m), lambda i: (0, 0)),
        ],
        out_specs=[
            pl.BlockSpec((ROW_BLK, m), lambda i: (i, 0)),
            pl.BlockSpec((ROW_BLK, m), lambda i: (i, 0)),
        ],
        out_shape=[
            jax.ShapeDtypeStruct((n, m), jnp.float32),
            jax.ShapeDtypeStruct((n, m), jnp.float32),
        ],
    )(a, w1, w2)


def _prep2_body(h_ref, wl_ref, wr_ref, att_ref, s_ref, d_ref):
    hh = h_ref[...]
    xl = jnp.dot(hh, wl_ref[...], preferred_element_type=jnp.float32)
    xr = jnp.dot(hh, wr_ref[...], preferred_element_type=jnp.float32)
    v = xl + xr
    t = jnp.maximum(v, 0.2 * v)
    mt = jnp.sum(t * att_ref[...], axis=1, keepdims=True)
    b = xl.shape[0]
    ones = jnp.ones((b, 1), jnp.float32)
    zer = jnp.zeros((b, W2 - C2 - 1), jnp.float32)
    s_ref[...] = jnp.concatenate([xl, ones, zer], axis=1)
    d_ref[...] = jnp.concatenate([xr, mt, zer], axis=1)


def _prep2(h, Wl2, Wr2, att2):
    return pl.pallas_call(
        _prep2_body,
        grid=(N // ROW_BLK,),
        in_specs=[
            pl.BlockSpec((ROW_BLK, D1), lambda i: (i, 0)),
            pl.BlockSpec((D1, C2), lambda i: (0, 0)),
            pl.BlockSpec((D1, C2), lambda i: (0, 0)),
            pl.BlockSpec((1, C2), lambda i: (0, 0)),
        ],
        out_specs=[
            pl.BlockSpec((ROW_BLK, W2), lambda i: (i, 0)),
            pl.BlockSpec((ROW_BLK, W2), lambda i: (i, 0)),
        ],
        out_shape=[
            jax.ShapeDtypeStruct((N, W2), jnp.float32),
            jax.ShapeDtypeStruct((N, W2), jnp.float32),
        ],
    )(h, Wl2, Wr2, att2)


def _pool_body(a0_ref, a1_ref, s_ref, b2_ref, bf_ref, wout_ref, bout_ref,
               out_ref, pacc):
    i = pl.program_id(0)
    ngrid = pl.num_programs(0)
    a0 = a0_ref[...]
    a1 = a1_ref[...]
    sc = s_ref[...]
    num = a0[:, :C2] + a1[:, :C2] - sc[:, :C2]
    den = a0[:, C2:C2 + 1] + a1[:, C2:C2 + 1] - 1.0
    h2 = num / (den + 1e-16) + b2_ref[...]
    onehot = jnp.where(
        bf_ref[...] == lax.broadcasted_iota(jnp.int32, (1, G), 1).astype(jnp.float32),
        1.0, 0.0)
    hon = jnp.concatenate([h2, jnp.ones((h2.shape[0], 1), jnp.float32)], axis=1)
    p = lax.dot_general(onehot, hon, (((0,), (0,)), ((), ())),
                        preferred_element_type=jnp.float32)

    @pl.when(i == 0)
    def _():
        pacc[...] = p

    @pl.when(i > 0)
    def _():
        pacc[...] = pacc[...] + p

    @pl.when(i == ngrid - 1)
    def _():
        s = pacc[...]
        pooled = s[:, :C2] / jnp.maximum(s[:, C2:C2 + 1], 1.0)
        lg = jnp.dot(pooled, wout_ref[...],
                     preferred_element_type=jnp.float32) + bout_ref[...]
        m = jnp.max(lg, axis=1, keepdims=True)
        ez = jnp.exp(lg - m)
        out_ref[...] = (lg - m) - jnp.log(jnp.sum(ez, axis=1, keepdims=True))


def _pool(acc0, acc1, srcT2, b2, batchf, Wout, bout):
    return pl.pallas_call(
        _pool_body,
        grid=(N // ROW_BLK,),
        in_specs=[
            pl.BlockSpec((ROW_BLK, W2), lambda i: (i, 0)),
            pl.BlockSpec((ROW_BLK, W2), lambda i: (i, 0)),
            pl.BlockSpec((ROW_BLK, W2), lambda i: (i, 0)),
            pl.BlockSpec((1, C2), lambda i: (0, 0)),
            pl.BlockSpec((ROW_BLK, 1), lambda i: (i, 0)),
            pl.BlockSpec((C2, NCLS), lambda i: (0, 0)),
            pl.BlockSpec((1, NCLS), lambda i: (0, 0)),
        ],
        out_specs=pl.BlockSpec((G, NCLS), lambda i: (0, 0)),
        out_shape=jax.ShapeDtypeStruct((G, NCLS), jnp.float32),
        scratch_shapes=[pltpu.VMEM((G, C2 + 1), jnp.float32)],
    )(acc0, acc1, srcT2, b2, batchf, Wout, bout)


def _prep1_body(x_ref, wl_ref, wr_ref, att_ref, i_ref, s_ref, d_ref):
    a = x_ref[...]
    xl = jnp.dot(a, wl_ref[...], preferred_element_type=jnp.float32)
    xr = jnp.dot(a, wr_ref[...], preferred_element_type=jnp.float32)
    v = xl + xr
    t = jnp.maximum(v, 0.2 * v) * att_ref[...]
    r512 = lax.broadcasted_iota(jnp.int32, (D1, 32), 0)
    c32 = lax.broadcasted_iota(jnp.int32, (D1, 32), 1)
    sel = jnp.where((c32 < H1) & (r512 // C1 == c32), 1.0, 0.0)
    mt32 = jnp.dot(t, sel, preferred_element_type=jnp.float32)
    b = a.shape[0]
    i16 = lax.broadcasted_iota(jnp.int32, (b, 16), 1)
    ones8 = jnp.where(i16 < H1, 1.0, 0.0)
    r16 = lax.broadcasted_iota(jnp.int32, (D1, D1), 0)
    c16b = lax.broadcasted_iota(jnp.int32, (D1, D1), 1)
    blk = (r16 // 32 == c16b // 32)
    pos = c16b % 32
    ch = r16 % 32
    perm = jnp.where(
        blk & (jnp.where(pos < 16, 2 * pos, 2 * (pos - 16) + 1) == ch),
        1.0, 0.0)
    xlp = jnp.dot(xl, perm, preferred_element_type=jnp.float32)
    i_ref[:, :D1] = xlp
    i_ref[:, D1:W1] = ones8
    s_ref[...] = xl.astype(jnp.bfloat16)
    d_ref[:, :D1] = xr.astype(jnp.bfloat16)
    d_ref[:, D1:WD] = mt32.astype(jnp.bfloat16)


def _prep1(xpad, Wl1, Wr1, att1f):
    return pl.pallas_call(
        _prep1_body,
        grid=(NP // ROW_BLK1,),
        in_specs=[
            pl.BlockSpec((ROW_BLK1, F_IN), lambda i: (i, 0)),
            pl.BlockSpec((F_IN, D1), lambda i: (0, 0)),
            pl.BlockSpec((F_IN, D1), lambda i: (0, 0)),
            pl.BlockSpec((1, D1), lambda i: (0, 0)),
        ],
        out_specs=[
            pl.BlockSpec((ROW_BLK1, W1), lambda i: (i, 0)),
            pl.BlockSpec((ROW_BLK1, WS), lambda i: (i, 0)),
            pl.BlockSpec((ROW_BLK1, WD), lambda i: (i, 0)),
        ],
        out_shape=[
            jax.ShapeDtypeStruct((NP, W1), jnp.float32),
            jax.ShapeDtypeStruct((NP, WS), jnp.bfloat16),
            jax.ShapeDtypeStruct((NP, WD), jnp.bfloat16),
        ],
    )(xpad, Wl1, Wr1, att1f)


_HI = jnp.int32(-65536)


def _l1_edge_body(init_hbm, src_hbm, dst_hbm, att_hbm, esrc_hbm, edst_hbm,
                  out_hbm, acc_sp,
                  srowsA, drowsA, orowsA, srowsB, drowsB, orowsB,
                  src_c, dst_c, esb, edb,
                  sidxA, didxA, gidxA, sidxB, didxB, gidxB, attv,
                  sgA, sgB, ssA, ssB):
    cid = lax.axis_index("c")
    sid = lax.axis_index("s")
    pltpu.sync_copy(att_hbm, attv)
    att_r = [attv[pl.ds(j * 16, 16)] for j in range(D1 // 16)]
    iota = lax.iota(jnp.int32, 16)
    rows_pt = QN // NSUB
    bufs = ((srowsA, drowsA, orowsA, sidxA, didxA, gidxA, sgA, ssA),
            (srowsB, drowsB, orowsB, sidxB, didxB, gidxB, sgB, ssB))

    def unpack2(ref, e, j):
        u = ref[e, pl.ds(j * 16, 16)]
        ev = plsc.bitcast(jnp.left_shift(u, 16), jnp.float32)
        od = plsc.bitcast(u & _HI, jnp.float32)
        return ev, od

    for p in range(NPASS1):
        q = p * NSC + cid
        lo = q * QN
        pltpu.sync_copy(init_hbm.at[pl.ds(lo + sid * rows_pt, rows_pt)],
                        acc_sp.at[pl.ds(sid * rows_pt, rows_pt)])
        plsc.subcore_barrier()

        def prep_fire(c, k, buf):
            srows, drows, orows, sidxb, didxb, gidxb, sg, _ = buf
            base = c * KB1
            for v in range(KB1 // 16):
                lanes = base + v * 16 + iota
                ok = lanes < k
                sv = jnp.where(ok, src_c[pl.ds(base + v * 16, 16)], 0)
                dv = jnp.where(ok, dst_c[pl.ds(base + v * 16, 16)], 0)
                sidxb[pl.ds(v * 16, 16)] = sv
                didxb[pl.ds(v * 16, 16)] = dv
                gidxb[pl.ds(v * 16, 16)] = dv + lo
            pltpu.async_copy(src_hbm.at[sidxb], srows, sg)
            pltpu.async_copy(dst_hbm.at[gidxb], drows, sg)

        def wait_gathers(buf):
            srows, drows, orows, sidxb, didxb, gidxb, sg, _ = buf
            pltpu.make_async_copy(src_hbm.at[sidxb], srows, sg).wait()
            pltpu.make_async_copy(dst_hbm.at[gidxb], drows, sg).wait()

        def wait_scatter(buf):
            _, _, orows, _, didxb, _, _, ss = buf
            pltpu.make_async_copy(orows, acc_sp.at[didxb], ss).wait()

        def compute_scatter(c, k, buf):
            srows, drows, orows, sidxb, didxb, gidxb, _, ss = buf
            base = c * KB1

            def edge(e):
                ut = drows[e, pl.ds(D1 // 2, 16)]
                mte = plsc.bitcast(jnp.left_shift(ut, 16), jnp.float32)
                mto = plsc.bitcast(ut & _HI, jnp.float32)
                valid = base + e < k
                ex16 = jnp.zeros((16,), jnp.float32)
                for h in range(H1):
                    pacc = jnp.zeros((16,), jnp.float32)
                    avs = []
                    for j in (2 * h, 2 * h + 1):
                        ae, ao = unpack2(srows, e, j)
                        be, bo = unpack2(drows, e, j)
                        avs.append((ae, ao))
                        ve = ae + be
                        te = jnp.maximum(ve, 0.2 * ve)
                        pacc = pacc + te * att_r[2 * j]
                        vo = ao + bo
                        to = jnp.maximum(vo, 0.2 * vo)
                        pacc = pacc + to * att_r[2 * j + 1]
                    mtx = mte if (h % 2 == 0) else mto
                    qv = pacc - jnp.where(iota == (h // 2), mtx, 0.0)
                    lz = jnp.sum(qv)
                    exh = jnp.exp(jnp.zeros((16,), jnp.float32) + lz)
                    exh = jnp.where(valid, exh, jnp.zeros((16,), jnp.float32))
                    for bi, j in enumerate((2 * h, 2 * h + 1)):
                        ae, ao = avs[bi]
                        orows[e, pl.ds(j * 32, 16)] = ae * exh
                        orows[e, pl.ds(j * 32 + 16, 16)] = ao * exh
                    ex16 = ex16 + jnp.where(iota == h, exh, 0.0)
                orows[e, pl.ds(D1, 16)] = ex16

            def ebody(e2, carry3):
                edge(2 * e2)
                edge(2 * e2 + 1)
                return carry3

            lax.fori_loop(0, KB1 // 2, ebody, 0)
            pltpu.async_copy(orows, acc_sp.at[didxb], ss, add=True)

        def scan_chunk(c, carry):
            off = sid * EPT1 + c * SCAN_CH
            pltpu.sync_copy(esrc_hbm.at[pl.ds(off, SCAN_CH)], esb)
            pltpu.sync_copy(edst_hbm.at[pl.ds(off, SCAN_CH)], edb)

            def vbody(vv, k2):
                d = edb[pl.ds(vv * 16, 16)]
                dl = d - lo
                msk = (dl >= 0) & (dl < QN)
                s = esb[pl.ds(vv * 16, 16)]
                plsc.store_compressed(dst_c.at[pl.ds(k2, 16)], dl, mask=msk)
                plsc.store_compressed(src_c.at[pl.ds(k2, 16)], s, mask=msk)
                cnt = plsc.all_reduce_population_count(msk)
                return k2 + jnp.max(cnt)

            k = lax.fori_loop(0, SCAN_CH // 16, vbody, jnp.int32(0))
            nch = (k + KB1 - 1) // KB1

            @pl.when(nch > 0)
            def _():
                prep_fire(0, k, bufs[0])

            def pair(c2, carry2):
                ce = 2 * c2
                co = ce + 1

                @pl.when(co < nch)
                def _():
                    @pl.when(c2 > 0)
                    def _():
                        wait_scatter(bufs[1])
                    prep_fire(co, k, bufs[1])

                @pl.when(ce < nch)
                def _():
                    wait_gathers(bufs[0])
                    compute_scatter(ce, k, bufs[0])

                @pl.when(co < nch)
                def _():
                    wait_gathers(bufs[1])
                    compute_scatter(co, k, bufs[1])

                @pl.when(ce + 2 < nch)
                def _():
                    wait_scatter(bufs[0])
                    prep_fire(ce + 2, k, bufs[0])

                return carry2

            lax.fori_loop(0, (nch + 1) // 2, pair, 0)

            @pl.when(nch >= 1)
            def _():
                wait_scatter(bufs[0])

            @pl.when(nch >= 2)
            def _():
                wait_scatter(bufs[1])

            return carry

        lax.fori_loop(0, NSCAN, scan_chunk, 0)
        plsc.subcore_barrier()
        pltpu.sync_copy(acc_sp.at[pl.ds(sid * rows_pt, rows_pt)],
                        out_hbm.at[pl.ds(lo + sid * rows_pt, rows_pt)])
        plsc.subcore_barrier()


def _l1_edges(initT1, srcT1, dstT1, att1f, esrc, edst):
    k = functools.partial(
        pl.kernel,
        out_type=jax.ShapeDtypeStruct((NP, W1), jnp.float32),
        mesh=_SC_MESH,
        compiler_params=pltpu.CompilerParams(
            needs_layout_passes=False, use_tc_tiling_on_sc=False),
        scratch_types=[
            pltpu.VMEM_SHARED((QN, W1), jnp.float32),
            pltpu.VMEM((KB1, WS // 2), jnp.int32),
            pltpu.VMEM((KB1, WD // 2), jnp.int32),
            pltpu.VMEM((KB1, W1), jnp.float32),
            pltpu.VMEM((KB1, WS // 2), jnp.int32),
            pltpu.VMEM((KB1, WD // 2), jnp.int32),
            pltpu.VMEM((KB1, W1), jnp.float32),
            pltpu.VMEM((CAP1,), jnp.int32),
            pltpu.VMEM((CAP1,), jnp.int32),
            pltpu.VMEM((SCAN_CH,), jnp.int32),
            pltpu.VMEM((SCAN_CH,), jnp.int32),
            pltpu.VMEM((KB1,), jnp.int32),
            pltpu.VMEM((KB1,), jnp.int32),
            pltpu.VMEM((KB1,), jnp.int32),
            pltpu.VMEM((KB1,), jnp.int32),
            pltpu.VMEM((KB1,), jnp.int32),
            pltpu.VMEM((KB1,), jnp.int32),
            pltpu.VMEM((D1,), jnp.float32),
            pltpu.SemaphoreType.DMA,
            pltpu.SemaphoreType.DMA,
            pltpu.SemaphoreType.DMA,
            pltpu.SemaphoreType.DMA,
        ],
    )(_l1_edge_body)
    return k(initT1, srcT1, dstT1, att1f, esrc, edst)


def _fin1_body(acc_ref, b1_ref, h_ref):
    a = acc_ref[...]
    num = a[:, :D1]
    den = a[:, D1:D1 + H1]
    r8 = lax.broadcasted_iota(jnp.int32, (H1, D1), 0)
    c512 = lax.broadcasted_iota(jnp.int32, (H1, D1), 1)
    sel = jnp.where(c512 // C1 == r8, 1.0, 0.0)
    den_b = jnp.dot(den, sel, preferred_element_type=jnp.float32)
    v = num / (den_b + 1e-16) + b1_ref[...]
    h_ref[...] = jnp.where(v > 0, v, jnp.exp(jnp.minimum(v, 0.0)) - 1.0)


def _fin1(accL1, b1):
    return pl.pallas_call(
        _fin1_body,
        grid=(N // ROW_BLK,),
        in_specs=[
            pl.BlockSpec((ROW_BLK, W1), lambda i: (i, 0)),
            pl.BlockSpec((1, D1), lambda i: (0, 0)),
        ],
        out_specs=pl.BlockSpec((ROW_BLK, D1), lambda i: (i, 0)),
        out_shape=jax.ShapeDtypeStruct((N, D1), jnp.float32),
    )(accL1, b1)


ROWS_PT = 624
ROWS_TAIL = N - ROWS_PT * NSUB


def _l2_edge_body(src_hbm, dst_hbm, att_hbm, esrc_hbm, edst_hbm, out_hbm,
                  acc_sp, srowsA, drowsA, srowsB, drowsB,
                  sidxA, didxA, sidxB, didxB, attv,
                  sgA, sgB, ssA, ssB):
    cid = lax.axis_index("c")
    sid = lax.axis_index("s")
    w = sid * NSC + cid
    start = sid * ROWS_PT

    pltpu.sync_copy(att_hbm, attv)
    pltpu.sync_copy(src_hbm.at[pl.ds(start, ROWS_PT)],
                    acc_sp.at[pl.ds(start, ROWS_PT)])

    @pl.when(sid == 0)
    def _():
        pltpu.sync_copy(src_hbm.at[pl.ds(ROWS_PT * NSUB, ROWS_TAIL)],
                        acc_sp.at[pl.ds(ROWS_PT * NSUB, ROWS_TAIL)])

    plsc.subcore_barrier()

    e0 = w * EPT2
    att_r = [attv[pl.ds(j * 16, 16)] for j in range(C2 // 16)]
    bufs = ((srowsA, drowsA, sidxA, didxA, sgA, ssA),
            (srowsB, drowsB, sidxB, didxB, sgB, ssB))

    def prep_fire(c, buf):
        srows, drows, sidxb, didxb, sg, _ = buf
        off = e0 + c * KB2
        pltpu.sync_copy(esrc_hbm.at[pl.ds(off, KB2)], sidxb)
        pltpu.sync_copy(edst_hbm.at[pl.ds(off, KB2)], didxb)
        pltpu.async_copy(src_hbm.at[sidxb], srows, sg)
        pltpu.async_copy(dst_hbm.at[didxb], drows, sg)

    def wait_gathers(buf):
        srows, drows, sidxb, didxb, sg, _ = buf
        pltpu.make_async_copy(src_hbm.at[sidxb], srows, sg).wait()
        pltpu.make_async_copy(dst_hbm.at[didxb], drows, sg).wait()

    def wait_scatter(buf):
        srows, _, _, didxb, _, ss = buf
        pltpu.make_async_copy(srows, acc_sp.at[didxb], ss).wait()

    def compute_scatter(c, buf):
        srows, drows, sidxb, didxb, _, ss = buf

        def edge(e):
            p = jnp.zeros((16,), jnp.float32)
            avs = []
            for j in range(C2 // 16):
                a = srows[e, pl.ds(j * 16, 16)]
                avs.append(a)
                b = drows[e, pl.ds(j * 16, 16)]
                v = a + b
                t = jnp.maximum(v, 0.2 * v)
                p = p + t * att_r[j]
            logit = jnp.sum(p)
            mt = jnp.sum(drows[e, pl.ds(C2, 16)])
            exv = jnp.exp(jnp.zeros((16,), jnp.float32) + (logit - mt))
            for j in range(C2 // 16):
                srows[e, pl.ds(j * 16, 16)] = avs[j] * exv
            srows[e, pl.ds(C2, 16)] = srows[e, pl.ds(C2, 16)] * exv

        def ebody(e2, carry2):
            edge(2 * e2)
            edge(2 * e2 + 1)
            return carry2

        lax.fori_loop(0, KB2 // 2, ebody, 0)
        pltpu.async_copy(srows, acc_sp.at[didxb], ss, add=True)

    nch = NCH2
    prep_fire(0, bufs[0])

    def pair(c2, carry):
        ce = 2 * c2
        co = ce + 1

        @pl.when(co < nch)
        def _():
            @pl.when(c2 > 0)
            def _():
                wait_scatter(bufs[1])
            prep_fire(co, bufs[1])

        @pl.when(ce < nch)
        def _():
            wait_gathers(bufs[0])
            compute_scatter(ce, bufs[0])

        @pl.when(co < nch)
        def _():
            wait_gathers(bufs[1])
            compute_scatter(co, bufs[1])

        @pl.when(ce + 2 < nch)
        def _():
            wait_scatter(bufs[0])
            prep_fire(ce + 2, bufs[0])

        return carry

    lax.fori_loop(0, (nch + 1) // 2, pair, 0)
    wait_scatter(bufs[0])
    wait_scatter(bufs[1])
    plsc.subcore_barrier()
    pltpu.sync_copy(acc_sp.at[pl.ds(start, ROWS_PT)],
                    out_hbm.at[cid, pl.ds(start, ROWS_PT)])

    @pl.when(sid == 0)
    def _():
        pltpu.sync_copy(acc_sp.at[pl.ds(ROWS_PT * NSUB, ROWS_TAIL)],
                        out_hbm.at[cid, pl.ds(ROWS_PT * NSUB, ROWS_TAIL)])


def _l2_edges(srcT2, dstT2, att2f, esrc, edst):
    k = functools.partial(
        pl.kernel,
        out_type=jax.ShapeDtypeStruct((NSC, N, W2), jnp.float32),
        mesh=_SC_MESH,
        compiler_params=pltpu.CompilerParams(
            needs_layout_passes=False, use_tc_tiling_on_sc=False),
        scratch_types=[
            pltpu.VMEM_SHARED((N, W2), jnp.float32),
            pltpu.VMEM((KB2, W2), jnp.float32),
            pltpu.VMEM((KB2, W2), jnp.float32),
            pltpu.VMEM((KB2, W2), jnp.float32),
            pltpu.VMEM((KB2, W2), jnp.float32),
            pltpu.VMEM((KB2,), jnp.int32),
            pltpu.VMEM((KB2,), jnp.int32),
            pltpu.VMEM((KB2,), jnp.int32),
            pltpu.VMEM((KB2,), jnp.int32),
            pltpu.VMEM((C2,), jnp.float32),
            pltpu.SemaphoreType.DMA,
            pltpu.SemaphoreType.DMA,
            pltpu.SemaphoreType.DMA,
            pltpu.SemaphoreType.DMA,
        ],
    )(_l2_edge_body)
    return k(srcT2, dstT2, att2f, esrc, edst)


def kernel(x, Wl1, Wr1, att1, b1, Wl2, Wr2, att2, b2, Wout, bout, edge_index, batch):
    xpad = jnp.pad(x, ((0, NP - N), (0, 0)))
    att1f = att1.reshape(1, D1)
    initT1, srcT1, dstT1 = _prep1(xpad, Wl1, Wr1, att1f)
    pi = jnp.asarray(_PI)
    srcT1i = lax.bitcast_convert_type(
        srcT1.reshape(NP, WS // 2, 2), jnp.int32)
    dstT1i = lax.bitcast_convert_type(
        dstT1.reshape(NP, WD // 2, 2), jnp.int32)
    accL1 = _l1_edges(initT1, srcT1i, dstT1i, att1.reshape(D1)[pi],
                      edge_index[0], edge_index[1])
    h = _fin1(accL1, b1.reshape(1, D1)[:, pi])
    srcT2, dstT2 = _prep2(h, Wl2[pi, :], Wr2[pi, :], att2)
    acc = _l2_edges(srcT2, dstT2, att2.reshape(C2),
                    edge_index[0], edge_index[1])
    batchf = batch.astype(jnp.float32).reshape(N, 1)
    return _pool(acc[0], acc[1], srcT2, b2.reshape(1, C2), batchf,
                 Wout, bout.reshape(1, NCLS))


# speedup vs baseline: 1.9363x; 1.9363x over previous
import functools

import jax
import jax.numpy as jnp
from jax import lax
from jax.experimental import pallas as pl
from jax.experimental.pallas import tpu as pltpu
from jax.experimental.pallas import tpu_sc as plsc

N = 10000
E = 320000
F_IN = 128
H1 = 8
C1 = 64
D1 = H1 * C1
C2 = 64
G = 64
NCLS = 2

NSC = 2
NSUB = 16

ROW_BLK = 2000

NP = 10240
ROW_BLK1 = 2048
W1 = D1 + 16
NPASS1 = 4
QN = NP // (NSC * NPASS1)
EPT1 = E // NSUB
SCAN_CH = 4000
NSCAN = EPT1 // SCAN_CH
KB1 = 32
CAP1 = SCAN_CH + 16

W2 = 80
EPT2 = E // (NSC * NSUB)
KB2 = 80
NCH2 = EPT2 // KB2

_SC_MESH = plsc.VectorSubcoreMesh(
    core_axis_name="c", subcore_axis_name="s", num_cores=NSC, num_subcores=NSUB)


def _mm2_body(a_ref, w1_ref, w2_ref, o1_ref, o2_ref):
    a = a_ref[...]
    o1_ref[...] = jnp.dot(a, w1_ref[...], preferred_element_type=jnp.float32)
    o2_ref[...] = jnp.dot(a, w2_ref[...], preferred_element_type=jnp.float32)


def _mm2(a, w1, w2):
    n, k = a.shape
    m = w1.shape[1]
    return pl.pallas_call(
        _mm2_body,
        grid=(n // ROW_BLK,),
        in_specs=[
            pl.BlockSpec((ROW_BLK, k), lambda i: (i, 0)),
            pl.BlockSpec((k, m), lambda i: (0, 0)),
            pl.BlockSpec((k, m), lambda i: (0, 0)),
        ],
        out_specs=[
            pl.BlockSpec((ROW_BLK, m), lambda i: (i, 0)),
            pl.BlockSpec((ROW_BLK, m), lambda i: (i, 0)),
        ],
        out_shape=[
            jax.ShapeDtypeStruct((n, m), jnp.float32),
            jax.ShapeDtypeStruct((n, m), jnp.float32),
        ],
    )(a, w1, w2)


def _prep2_body(h_ref, wl_ref, wr_ref, att_ref, s_ref, d_ref):
    hh = h_ref[...]
    xl = jnp.dot(hh, wl_ref[...], preferred_element_type=jnp.float32)
    xr = jnp.dot(hh, wr_ref[...], preferred_element_type=jnp.float32)
    v = xl + xr
    t = jnp.maximum(v, 0.2 * v)
    mt = jnp.sum(t * att_ref[...], axis=1, keepdims=True)
    b = xl.shape[0]
    ones = jnp.ones((b, 1), jnp.float32)
    zer = jnp.zeros((b, W2 - C2 - 1), jnp.float32)
    s_ref[...] = jnp.concatenate([xl, ones, zer], axis=1)
    d_ref[...] = jnp.concatenate([xr, mt, zer], axis=1)


def _prep2(h, Wl2, Wr2, att2):
    return pl.pallas_call(
        _prep2_body,
        grid=(N // ROW_BLK,),
        in_specs=[
            pl.BlockSpec((ROW_BLK, D1), lambda i: (i, 0)),
            pl.BlockSpec((D1, C2), lambda i: (0, 0)),
            pl.BlockSpec((D1, C2), lambda i: (0, 0)),
            pl.BlockSpec((1, C2), lambda i: (0, 0)),
        ],
        out_specs=[
            pl.BlockSpec((ROW_BLK, W2), lambda i: (i, 0)),
            pl.BlockSpec((ROW_BLK, W2), lambda i: (i, 0)),
        ],
        out_shape=[
            jax.ShapeDtypeStruct((N, W2), jnp.float32),
            jax.ShapeDtypeStruct((N, W2), jnp.float32),
        ],
    )(h, Wl2, Wr2, att2)


def _pool_body(a0_ref, a1_ref, s_ref, b2_ref, bf_ref, wout_ref, bout_ref,
               out_ref, pacc):
    i = pl.program_id(0)
    ngrid = pl.num_programs(0)
    a0 = a0_ref[...]
    a1 = a1_ref[...]
    sc = s_ref[...]
    num = a0[:, :C2] + a1[:, :C2] - sc[:, :C2]
    den = a0[:, C2:C2 + 1] + a1[:, C2:C2 + 1] - 1.0
    h2 = num / (den + 1e-16) + b2_ref[...]
    onehot = jnp.where(
        bf_ref[...] == lax.broadcasted_iota(jnp.int32, (1, G), 1).astype(jnp.float32),
        1.0, 0.0)
    hon = jnp.concatenate([h2, jnp.ones((h2.shape[0], 1), jnp.float32)], axis=1)
    p = lax.dot_general(onehot, hon, (((0,), (0,)), ((), ())),
                        preferred_element_type=jnp.float32)

    @pl.when(i == 0)
    def _():
        pacc[...] = p

    @pl.when(i > 0)
    def _():
        pacc[...] = pacc[...] + p

    @pl.when(i == ngrid - 1)
    def _():
        s = pacc[...]
        pooled = s[:, :C2] / jnp.maximum(s[:, C2:C2 + 1], 1.0)
        lg = jnp.dot(pooled, wout_ref[...],
                     preferred_element_type=jnp.float32) + bout_ref[...]
        m = jnp.max(lg, axis=1, keepdims=True)
        ez = jnp.exp(lg - m)
        out_ref[...] = (lg - m) - jnp.log(jnp.sum(ez, axis=1, keepdims=True))


def _pool(acc0, acc1, srcT2, b2, batchf, Wout, bout):
    return pl.pallas_call(
        _pool_body,
        grid=(N // ROW_BLK,),
        in_specs=[
            pl.BlockSpec((ROW_BLK, W2), lambda i: (i, 0)),
            pl.BlockSpec((ROW_BLK, W2), lambda i: (i, 0)),
            pl.BlockSpec((ROW_BLK, W2), lambda i: (i, 0)),
            pl.BlockSpec((1, C2), lambda i: (0, 0)),
            pl.BlockSpec((ROW_BLK, 1), lambda i: (i, 0)),
            pl.BlockSpec((C2, NCLS), lambda i: (0, 0)),
            pl.BlockSpec((1, NCLS), lambda i: (0, 0)),
        ],
        out_specs=pl.BlockSpec((G, NCLS), lambda i: (0, 0)),
        out_shape=jax.ShapeDtypeStruct((G, NCLS), jnp.float32),
        scratch_shapes=[pltpu.VMEM((G, C2 + 1), jnp.float32)],
    )(acc0, acc1, srcT2, b2, batchf, Wout, bout)


def _prep1_body(x_ref, wl_ref, wr_ref, att_ref, s_ref, d_ref):
    a = x_ref[...]
    xl = jnp.dot(a, wl_ref[...], preferred_element_type=jnp.float32)
    xr = jnp.dot(a, wr_ref[...], preferred_element_type=jnp.float32)
    v = xl + xr
    t = jnp.maximum(v, 0.2 * v) * att_ref[...]
    r512 = lax.broadcasted_iota(jnp.int32, (D1, 16), 0)
    c16 = lax.broadcasted_iota(jnp.int32, (D1, 16), 1)
    sel = jnp.where((c16 < H1) & (r512 // C1 == c16), 1.0, 0.0)
    mt16 = jnp.dot(t, sel, preferred_element_type=jnp.float32)
    b = a.shape[0]
    i16 = lax.broadcasted_iota(jnp.int32, (b, 16), 1)
    ones8 = jnp.where(i16 < H1, 1.0, 0.0)
    s_ref[:, :D1] = xl
    s_ref[:, D1:W1] = ones8
    d_ref[:, :D1] = xr
    d_ref[:, D1:W1] = mt16


def _prep1(xpad, Wl1, Wr1, att1f):
    return pl.pallas_call(
        _prep1_body,
        grid=(NP // ROW_BLK1,),
        in_specs=[
            pl.BlockSpec((ROW_BLK1, F_IN), lambda i: (i, 0)),
            pl.BlockSpec((F_IN, D1), lambda i: (0, 0)),
            pl.BlockSpec((F_IN, D1), lambda i: (0, 0)),
            pl.BlockSpec((1, D1), lambda i: (0, 0)),
        ],
        out_specs=[
            pl.BlockSpec((ROW_BLK1, W1), lambda i: (i, 0)),
            pl.BlockSpec((ROW_BLK1, W1), lambda i: (i, 0)),
        ],
        out_shape=[
            jax.ShapeDtypeStruct((NP, W1), jnp.float32),
            jax.ShapeDtypeStruct((NP, W1), jnp.float32),
        ],
    )(xpad, Wl1, Wr1, att1f)


def _l1_edge_body(src_hbm, dst_hbm, att_hbm, esrc_hbm, edst_hbm, out_hbm,
                  acc_sp,
                  srowsA, drowsA, srowsB, drowsB,
                  src_c, dst_c, esb, edb,
                  sidxA, didxA, gidxA, sidxB, didxB, gidxB, attv,
                  sgA, sgB, ssA, ssB):
    cid = lax.axis_index("c")
    sid = lax.axis_index("s")
    pltpu.sync_copy(att_hbm, attv)
    att_r = [attv[pl.ds(j * 16, 16)] for j in range(D1 // 16)]
    iota = lax.iota(jnp.int32, 16)
    rows_pt = QN // NSUB
    bufs = ((srowsA, drowsA, sidxA, didxA, gidxA, sgA, ssA),
            (srowsB, drowsB, sidxB, didxB, gidxB, sgB, ssB))

    for p in range(NPASS1):
        q = p * NSC + cid
        lo = q * QN
        pltpu.sync_copy(src_hbm.at[pl.ds(lo + sid * rows_pt, rows_pt)],
                        acc_sp.at[pl.ds(sid * rows_pt, rows_pt)])
        plsc.subcore_barrier()

        def prep_fire(c, k, buf):
            srows, drows, sidxb, didxb, gidxb, sg, _ = buf
            base = c * KB1
            for v in range(KB1 // 16):
                lanes = base + v * 16 + iota
                ok = lanes < k
                sv = jnp.where(ok, src_c[pl.ds(base + v * 16, 16)], 0)
                dv = jnp.where(ok, dst_c[pl.ds(base + v * 16, 16)], 0)
                sidxb[pl.ds(v * 16, 16)] = sv
                didxb[pl.ds(v * 16, 16)] = dv
                gidxb[pl.ds(v * 16, 16)] = dv + lo
            pltpu.async_copy(src_hbm.at[sidxb], srows, sg)
            pltpu.async_copy(dst_hbm.at[gidxb], drows, sg)

        def wait_gathers(buf):
            srows, drows, sidxb, didxb, gidxb, sg, _ = buf
            pltpu.make_async_copy(src_hbm.at[sidxb], srows, sg).wait()
            pltpu.make_async_copy(dst_hbm.at[gidxb], drows, sg).wait()

        def wait_scatter(buf):
            srows, _, _, didxb, _, _, ss = buf
            pltpu.make_async_copy(srows, acc_sp.at[didxb], ss).wait()

        def compute_scatter(c, k, buf):
            srows, drows, sidxb, didxb, gidxb, _, ss = buf
            base = c * KB1

            def edge(e):
                mtv = drows[e, pl.ds(D1, 16)]
                valid = base + e < k
                ex16 = jnp.zeros((16,), jnp.float32)
                for h in range(H1):
                    pacc = jnp.zeros((16,), jnp.float32)
                    avs = []
                    for j in range(4):
                        jj = h * 4 + j
                        a = srows[e, pl.ds(jj * 16, 16)]
                        avs.append(a)
                        b = drows[e, pl.ds(jj * 16, 16)]
                        vv2 = a + b
                        t = jnp.maximum(vv2, 0.2 * vv2)
                        pacc = pacc + t * att_r[jj]
                    qv = pacc - jnp.where(iota == h, mtv, 0.0)
                    lz = jnp.sum(qv)
                    exh = jnp.exp(jnp.zeros((16,), jnp.float32) + lz)
                    exh = jnp.where(valid, exh, jnp.zeros((16,), jnp.float32))
                    for j in range(4):
                        jj = h * 4 + j
                        srows[e, pl.ds(jj * 16, 16)] = avs[j] * exh
                    ex16 = ex16 + jnp.where(iota == h, exh, 0.0)
                srows[e, pl.ds(D1, 16)] = srows[e, pl.ds(D1, 16)] * ex16

            def ebody(e2, carry3):
                edge(2 * e2)
                edge(2 * e2 + 1)
                return carry3

            lax.fori_loop(0, KB1 // 2, ebody, 0)
            pltpu.async_copy(srows, acc_sp.at[didxb], ss, add=True)

        def scan_chunk(c, carry):
            off = sid * EPT1 + c * SCAN_CH
            pltpu.sync_copy(esrc_hbm.at[pl.ds(off, SCAN_CH)], esb)
            pltpu.sync_copy(edst_hbm.at[pl.ds(off, SCAN_CH)], edb)

            def vbody(vv, k2):
                d = edb[pl.ds(vv * 16, 16)]
                dl = d - lo
                msk = (dl >= 0) & (dl < QN)
                s = esb[pl.ds(vv * 16, 16)]
                plsc.store_compressed(dst_c.at[pl.ds(k2, 16)], dl, mask=msk)
                plsc.store_compressed(src_c.at[pl.ds(k2, 16)], s, mask=msk)
                cnt = plsc.all_reduce_population_count(msk)
                return k2 + jnp.max(cnt)

            k = lax.fori_loop(0, SCAN_CH // 16, vbody, jnp.int32(0))
            nch = (k + KB1 - 1) // KB1

            @pl.when(nch > 0)
            def _():
                prep_fire(0, k, bufs[0])

            def pair(c2, carry2):
                ce = 2 * c2
                co = ce + 1

                @pl.when(co < nch)
                def _():
                    @pl.when(c2 > 0)
                    def _():
                        wait_scatter(bufs[1])
                    prep_fire(co, k, bufs[1])

                @pl.when(ce < nch)
                def _():
                    wait_gathers(bufs[0])
                    compute_scatter(ce, k, bufs[0])

                @pl.when(co < nch)
                def _():
                    wait_gathers(bufs[1])
                    compute_scatter(co, k, bufs[1])

                @pl.when(ce + 2 < nch)
                def _():
                    wait_scatter(bufs[0])
                    prep_fire(ce + 2, k, bufs[0])

                return carry2

            lax.fori_loop(0, (nch + 1) // 2, pair, 0)

            @pl.when(nch >= 1)
            def _():
                wait_scatter(bufs[0])

            @pl.when(nch >= 2)
            def _():
                wait_scatter(bufs[1])

            return carry

        lax.fori_loop(0, NSCAN, scan_chunk, 0)
        plsc.subcore_barrier()
        pltpu.sync_copy(acc_sp.at[pl.ds(sid * rows_pt, rows_pt)],
                        out_hbm.at[pl.ds(lo + sid * rows_pt, rows_pt)])
        plsc.subcore_barrier()


def _l1_edges(srcT1, dstT1, att1f, esrc, edst):
    k = functools.partial(
        pl.kernel,
        out_type=jax.ShapeDtypeStruct((NP, W1), jnp.float32),
        mesh=_SC_MESH,
        compiler_params=pltpu.CompilerParams(
            needs_layout_passes=False, use_tc_tiling_on_sc=False),
        scratch_types=[
            pltpu.VMEM_SHARED((QN, W1), jnp.float32),
            pltpu.VMEM((KB1, W1), jnp.float32),
            pltpu.VMEM((KB1, W1), jnp.float32),
            pltpu.VMEM((KB1, W1), jnp.float32),
            pltpu.VMEM((KB1, W1), jnp.float32),
            pltpu.VMEM((CAP1,), jnp.int32),
            pltpu.VMEM((CAP1,), jnp.int32),
            pltpu.VMEM((SCAN_CH,), jnp.int32),
            pltpu.VMEM((SCAN_CH,), jnp.int32),
            pltpu.VMEM((KB1,), jnp.int32),
            pltpu.VMEM((KB1,), jnp.int32),
            pltpu.VMEM((KB1,), jnp.int32),
            pltpu.VMEM((KB1,), jnp.int32),
            pltpu.VMEM((KB1,), jnp.int32),
            pltpu.VMEM((KB1,), jnp.int32),
            pltpu.VMEM((D1,), jnp.float32),
            pltpu.SemaphoreType.DMA,
            pltpu.SemaphoreType.DMA,
            pltpu.SemaphoreType.DMA,
            pltpu.SemaphoreType.DMA,
        ],
    )(_l1_edge_body)
    return k(srcT1, dstT1, att1f, esrc, edst)


def _fin1_body(acc_ref, b1_ref, h_ref):
    a = acc_ref[...]
    num = a[:, :D1]
    den = a[:, D1:D1 + H1]
    r8 = lax.broadcasted_iota(jnp.int32, (H1, D1), 0)
    c512 = lax.broadcasted_iota(jnp.int32, (H1, D1), 1)
    sel = jnp.where(c512 // C1 == r8, 1.0, 0.0)
    den_b = jnp.dot(den, sel, preferred_element_type=jnp.float32)
    v = num / (den_b + 1e-16) + b1_ref[...]
    h_ref[...] = jnp.where(v > 0, v, jnp.exp(jnp.minimum(v, 0.0)) - 1.0)


def _fin1(accL1, b1):
    return pl.pallas_call(
        _fin1_body,
        grid=(N // ROW_BLK,),
        in_specs=[
            pl.BlockSpec((ROW_BLK, W1), lambda i: (i, 0)),
            pl.BlockSpec((1, D1), lambda i: (0, 0)),
        ],
        out_specs=pl.BlockSpec((ROW_BLK, D1), lambda i: (i, 0)),
        out_shape=jax.ShapeDtypeStruct((N, D1), jnp.float32),
    )(accL1, b1)


ROWS_PT = 624
ROWS_TAIL = N - ROWS_PT * NSUB


def _l2_edge_body(src_hbm, dst_hbm, att_hbm, esrc_hbm, edst_hbm, out_hbm,
                  acc_sp, srowsA, drowsA, srowsB, drowsB,
                  sidxA, didxA, sidxB, didxB, attv,
                  sgA, sgB, ssA, ssB):
    cid = lax.axis_index("c")
    sid = lax.axis_index("s")
    w = sid * NSC + cid
    start = sid * ROWS_PT

    pltpu.sync_copy(att_hbm, attv)
    pltpu.sync_copy(src_hbm.at[pl.ds(start, ROWS_PT)],
                    acc_sp.at[pl.ds(start, ROWS_PT)])

    @pl.when(sid == 0)
    def _():
        pltpu.sync_copy(src_hbm.at[pl.ds(ROWS_PT * NSUB, ROWS_TAIL)],
                        acc_sp.at[pl.ds(ROWS_PT * NSUB, ROWS_TAIL)])

    plsc.subcore_barrier()

    e0 = w * EPT2
    att_r = [attv[pl.ds(j * 16, 16)] for j in range(C2 // 16)]
    bufs = ((srowsA, drowsA, sidxA, didxA, sgA, ssA),
            (srowsB, drowsB, sidxB, didxB, sgB, ssB))

    def prep_fire(c, buf):
        srows, drows, sidxb, didxb, sg, _ = buf
        off = e0 + c * KB2
        pltpu.sync_copy(esrc_hbm.at[pl.ds(off, KB2)], sidxb)
        pltpu.sync_copy(edst_hbm.at[pl.ds(off, KB2)], didxb)
        pltpu.async_copy(src_hbm.at[sidxb], srows, sg)
        pltpu.async_copy(dst_hbm.at[didxb], drows, sg)

    def wait_gathers(buf):
        srows, drows, sidxb, didxb, sg, _ = buf
        pltpu.make_async_copy(src_hbm.at[sidxb], srows, sg).wait()
        pltpu.make_async_copy(dst_hbm.at[didxb], drows, sg).wait()

    def wait_scatter(buf):
        srows, _, _, didxb, _, ss = buf
        pltpu.make_async_copy(srows, acc_sp.at[didxb], ss).wait()

    def compute_scatter(c, buf):
        srows, drows, sidxb, didxb, _, ss = buf

        def edge(e):
            p = jnp.zeros((16,), jnp.float32)
            avs = []
            for j in range(C2 // 16):
                a = srows[e, pl.ds(j * 16, 16)]
                avs.append(a)
                b = drows[e, pl.ds(j * 16, 16)]
                v = a + b
                t = jnp.maximum(v, 0.2 * v)
                p = p + t * att_r[j]
            logit = jnp.sum(p)
            mt = jnp.sum(drows[e, pl.ds(C2, 16)])
            exv = jnp.exp(jnp.zeros((16,), jnp.float32) + (logit - mt))
            for j in range(C2 // 16):
                srows[e, pl.ds(j * 16, 16)] = avs[j] * exv
            srows[e, pl.ds(C2, 16)] = srows[e, pl.ds(C2, 16)] * exv

        def ebody(e2, carry2):
            edge(2 * e2)
            edge(2 * e2 + 1)
            return carry2

        lax.fori_loop(0, KB2 // 2, ebody, 0)
        pltpu.async_copy(srows, acc_sp.at[didxb], ss, add=True)

    nch = NCH2
    prep_fire(0, bufs[0])

    def pair(c2, carry):
        ce = 2 * c2
        co = ce + 1

        @pl.when(co < nch)
        def _():
            @pl.when(c2 > 0)
            def _():
                wait_scatter(bufs[1])
            prep_fire(co, bufs[1])

        @pl.when(ce < nch)
        def _():
            wait_gathers(bufs[0])
            compute_scatter(ce, bufs[0])

        @pl.when(co < nch)
        def _():
            wait_gathers(bufs[1])
            compute_scatter(co, bufs[1])

        @pl.when(ce + 2 < nch)
        def _():
            wait_scatter(bufs[0])
            prep_fire(ce + 2, bufs[0])

        return carry

    lax.fori_loop(0, (nch + 1) // 2, pair, 0)
    wait_scatter(bufs[0])
    wait_scatter(bufs[1])
    plsc.subcore_barrier()
    pltpu.sync_copy(acc_sp.at[pl.ds(start, ROWS_PT)],
                    out_hbm.at[cid, pl.ds(start, ROWS_PT)])

    @pl.when(sid == 0)
    def _():
        pltpu.sync_copy(acc_sp.at[pl.ds(ROWS_PT * NSUB, ROWS_TAIL)],
                        out_hbm.at[cid, pl.ds(ROWS_PT * NSUB, ROWS_TAIL)])


def _l2_edges(srcT2, dstT2, att2f, esrc, edst):
    k = functools.partial(
        pl.kernel,
        out_type=jax.ShapeDtypeStruct((NSC, N, W2), jnp.float32),
        mesh=_SC_MESH,
        compiler_params=pltpu.CompilerParams(
            needs_layout_passes=False, use_tc_tiling_on_sc=False),
        scratch_types=[
            pltpu.VMEM_SHARED((N, W2), jnp.float32),
            pltpu.VMEM((KB2, W2), jnp.float32),
            pltpu.VMEM((KB2, W2), jnp.float32),
            pltpu.VMEM((KB2, W2), jnp.float32),
            pltpu.VMEM((KB2, W2), jnp.float32),
            pltpu.VMEM((KB2,), jnp.int32),
            pltpu.VMEM((KB2,), jnp.int32),
            pltpu.VMEM((KB2,), jnp.int32),
            pltpu.VMEM((KB2,), jnp.int32),
            pltpu.VMEM((C2,), jnp.float32),
            pltpu.SemaphoreType.DMA,
            pltpu.SemaphoreType.DMA,
            pltpu.SemaphoreType.DMA,
            pltpu.SemaphoreType.DMA,
        ],
    )(_l2_edge_body)
    return k(srcT2, dstT2, att2f, esrc, edst)


def kernel(x, Wl1, Wr1, att1, b1, Wl2, Wr2, att2, b2, Wout, bout, edge_index, batch):
    xpad = jnp.pad(x, ((0, NP - N), (0, 0)))
    att1f = att1.reshape(1, D1)
    srcT1, dstT1 = _prep1(xpad, Wl1, Wr1, att1f)
    accL1 = _l1_edges(srcT1, dstT1, att1.reshape(D1),
                      edge_index[0], edge_index[1])
    h = _fin1(accL1, b1.reshape(1, D1))
    srcT2, dstT2 = _prep2(h, Wl2, Wr2, att2)
    acc = _l2_edges(srcT2, dstT2, att2.reshape(C2),
                    edge_index[0], edge_index[1])
    batchf = batch.astype(jnp.float32).reshape(N, 1)
    return _pool(acc[0], acc[1], srcT2, b2.reshape(1, C2), batchf,
                 Wout, bout.reshape(1, NCLS))
